# baseline (speedup 1.0000x reference)
import functools
import math

import jax
import jax.numpy as jnp
from jax import lax
from jax.experimental import pallas as pl
from jax.experimental.pallas import tpu as pltpu
from jax.experimental.pallas import tpu_sc as plsc

N_HID = 128
MAX_LEN = 240
PAD_LEN = 256
BATCH = 4096
HIST = 200

NC = 2
NS = 16
NW = NC * NS

B_TOTAL = BATCH * HIST
B_TC = 262144
B_SC = B_TOTAL - B_TC
B_PER_W = B_SC // NW
CHUNK = 128
N_CHUNKS = B_PER_W // CHUNK
NBUF = 4
N_GROUPS = N_CHUNKS // NBUF

TBLK = 2048
N_TBLK = B_TC // TBLK


def _fuse_tc_kernel(emb_ref, w_ref, b_ref, out_ref, hi_ref, lo_ref):
    e = emb_ref[...]
    w = w_ref[...]
    acc = lax.dot_general(
        e, w, (((1,), (1,)), ((), ())), preferred_element_type=jnp.float32
    )
    fused = acc + b_ref[...]
    out_ref[...] = fused
    hi = fused.astype(jnp.bfloat16)
    lo = (fused - hi.astype(jnp.float32)).astype(jnp.bfloat16)
    zpad = jnp.zeros((PAD_LEN - MAX_LEN, N_HID), jnp.bfloat16)
    hi_ref[...] = jnp.concatenate([hi, zpad], axis=0)
    lo_ref[...] = jnp.concatenate([lo, zpad], axis=0)


def _build_fused_table(emb_weight, lin_w, lin_b):
    return pl.pallas_call(
        _fuse_tc_kernel,
        out_shape=[
            jax.ShapeDtypeStruct((MAX_LEN, N_HID), jnp.float32),
            jax.ShapeDtypeStruct((PAD_LEN, N_HID), jnp.bfloat16),
            jax.ShapeDtypeStruct((PAD_LEN, N_HID), jnp.bfloat16),
        ],
    )(emb_weight, lin_w, lin_b.reshape(1, N_HID))


def _onehot_tc_kernel(idx_ref, hi_ref, lo_ref, out_ref):
    idv = idx_ref[0, 0, :]
    iot = lax.broadcasted_iota(jnp.int32, (TBLK, PAD_LEN), 1)
    oh = (idv[:, None] == iot).astype(jnp.bfloat16)
    acc = jnp.dot(oh, hi_ref[...], preferred_element_type=jnp.float32)
    acc += jnp.dot(oh, lo_ref[...], preferred_element_type=jnp.float32)
    out_ref[...] = acc


def _tc_onehot(idx_tc, hi, lo):
    idx3 = idx_tc.reshape(N_TBLK, 1, TBLK)
    return pl.pallas_call(
        _onehot_tc_kernel,
        grid=(N_TBLK,),
        in_specs=[
            pl.BlockSpec((1, 1, TBLK), lambda i: (i, 0, 0)),
            pl.BlockSpec((PAD_LEN, N_HID), lambda i: (0, 0)),
            pl.BlockSpec((PAD_LEN, N_HID), lambda i: (0, 0)),
        ],
        out_specs=pl.BlockSpec((TBLK, N_HID), lambda i: (i, 0)),
        out_shape=jax.ShapeDtypeStruct((B_TC, N_HID), jnp.float32),
    )(idx3, hi, lo)


def _gather_body(table_hbm, idx_hbm, out_hbm, table_v, idx_v, rows, gsems, wsems):
    wid = lax.axis_index("s") * NC + lax.axis_index("c")
    w_base = pl.multiple_of(wid * B_PER_W, CHUNK)

    @pl.when(lax.axis_index("s") == 0)
    def _stage():
        pltpu.sync_copy(table_hbm, table_v)

    plsc.subcore_barrier()
    pltpu.sync_copy(idx_hbm.at[pl.ds(w_base, B_PER_W)], idx_v)

    def start_gather(chunk, b):
        off = pl.multiple_of(chunk * CHUNK, CHUNK)
        pltpu.async_copy(table_v.at[idx_v.at[pl.ds(off, CHUNK)]], rows[b], gsems[b])

    def start_write(chunk, b):
        off = pl.multiple_of(w_base + chunk * CHUNK, CHUNK)
        pltpu.async_copy(rows[b], out_hbm.at[pl.ds(off, CHUNK)], wsems[b])

    def drain_gather(b):
        pltpu.make_async_copy(
            table_v.at[idx_v.at[pl.ds(0, CHUNK)]], rows[b], gsems[b]
        ).wait()

    def drain_write(b):
        pltpu.make_async_copy(
            rows[b], out_hbm.at[pl.ds(w_base, CHUNK)], wsems[b]
        ).wait()

    for b in range(NBUF):
        start_gather(b, b)

    @pl.loop(0, N_GROUPS - 1)
    def _group(g):
        base_chunk = g * NBUF
        for b in range(NBUF):
            drain_gather(b)
            start_write(base_chunk + b, b)
        for b in range(NBUF):
            drain_write(b)
            start_gather(base_chunk + NBUF + b, b)

    last = (N_GROUPS - 1) * NBUF
    for b in range(NBUF):
        drain_gather(b)
        start_write(last + b, b)
    for b in range(NBUF):
        drain_write(b)


@functools.partial(
    pl.kernel,
    out_type=jax.ShapeDtypeStruct((B_SC, N_HID), jnp.float32),
    mesh=plsc.VectorSubcoreMesh(core_axis_name="c", subcore_axis_name="s"),
    scratch_types=[
        pltpu.VMEM_SHARED((MAX_LEN, N_HID), jnp.float32),
        pltpu.VMEM((B_PER_W,), jnp.int32),
        [pltpu.VMEM((CHUNK, N_HID), jnp.float32) for _ in range(NBUF)],
        [pltpu.SemaphoreType.DMA for _ in range(NBUF)],
        [pltpu.SemaphoreType.DMA for _ in range(NBUF)],
    ],
)
def _sc_gather(table_hbm, idx_hbm, out_hbm, table_v, idx_v, rows, gsems, wsems):
    _gather_body(table_hbm, idx_hbm, out_hbm, table_v, idx_v, rows, gsems, wsems)


def kernel(time, emb_weight, lin_w, lin_b):
    fused, hi, lo = _build_fused_table(emb_weight, lin_w, lin_b)
    idx = time.reshape(B_TOTAL)
    out_sc = _sc_gather(fused, idx[B_TC:])
    out_tc = _tc_onehot(idx[:B_TC], hi, lo)
    out = jnp.concatenate([out_tc, out_sc], axis=0)
    return out.reshape(BATCH, HIST, N_HID)


# speedup vs baseline: 2.3887x; 2.3887x over previous
import functools
import math

import jax
import jax.numpy as jnp
from jax import lax
from jax.experimental import pallas as pl
from jax.experimental.pallas import tpu as pltpu
from jax.experimental.pallas import tpu_sc as plsc

N_HID = 128
MAX_LEN = 240
BATCH = 4096
HIST = 200

NC = 2
NS = 16
NW = NC * NS

B_TOTAL = BATCH * HIST
B_PER_W = B_TOTAL // NW
CHUNK = 128
N_CHUNKS = B_PER_W // CHUNK
NBUF = 4
N_GROUPS = N_CHUNKS // NBUF


def _fuse_tc_kernel(emb_ref, w_ref, b_ref, out_ref):
    e = emb_ref[...]
    w = w_ref[...]
    acc = lax.dot_general(
        e, w, (((1,), (1,)), ((), ())), preferred_element_type=jnp.float32
    )
    out_ref[...] = acc + b_ref[...]


def _build_fused_table(emb_weight, lin_w, lin_b):
    return pl.pallas_call(
        _fuse_tc_kernel,
        out_shape=jax.ShapeDtypeStruct((MAX_LEN, N_HID), jnp.float32),
    )(emb_weight, lin_w, lin_b.reshape(1, N_HID))


def _gather_body(table_hbm, idx_hbm, out_hbm, table_v, idx_v, rows, gsems, wsems):
    wid = lax.axis_index("s") * NC + lax.axis_index("c")
    w_base = pl.multiple_of(wid * B_PER_W, CHUNK)

    @pl.when(lax.axis_index("s") == 0)
    def _stage():
        pltpu.sync_copy(table_hbm, table_v)

    plsc.subcore_barrier()
    pltpu.sync_copy(idx_hbm.at[pl.ds(w_base, B_PER_W)], idx_v)

    def start_gather(chunk, b):
        off = pl.multiple_of(chunk * CHUNK, CHUNK)
        pltpu.async_copy(table_v.at[idx_v.at[pl.ds(off, CHUNK)]], rows[b], gsems[b])

    def start_write(chunk, b):
        off = pl.multiple_of(w_base + chunk * CHUNK, CHUNK)
        pltpu.async_copy(rows[b], out_hbm.at[pl.ds(off, CHUNK)], wsems[b])

    def drain_gather(b):
        pltpu.make_async_copy(
            table_v.at[idx_v.at[pl.ds(0, CHUNK)]], rows[b], gsems[b]
        ).wait()

    def drain_write(b):
        pltpu.make_async_copy(
            rows[b], out_hbm.at[pl.ds(w_base, CHUNK)], wsems[b]
        ).wait()

    for b in range(NBUF):
        start_gather(b, b)

    @pl.loop(0, N_GROUPS - 1)
    def _group(g):
        base_chunk = g * NBUF
        for b in range(NBUF):
            drain_gather(b)
            start_write(base_chunk + b, b)
        for b in range(NBUF):
            drain_write(b)
            start_gather(base_chunk + NBUF + b, b)

    last = (N_GROUPS - 1) * NBUF
    for b in range(NBUF):
        drain_gather(b)
        start_write(last + b, b)
    for b in range(NBUF):
        drain_write(b)


@functools.partial(
    pl.kernel,
    out_type=jax.ShapeDtypeStruct((B_TOTAL, N_HID), jnp.float32),
    mesh=plsc.VectorSubcoreMesh(core_axis_name="c", subcore_axis_name="s"),
    scratch_types=[
        pltpu.VMEM_SHARED((MAX_LEN, N_HID), jnp.float32),
        pltpu.VMEM((B_PER_W,), jnp.int32),
        [pltpu.VMEM((CHUNK, N_HID), jnp.float32) for _ in range(NBUF)],
        [pltpu.SemaphoreType.DMA for _ in range(NBUF)],
        [pltpu.SemaphoreType.DMA for _ in range(NBUF)],
    ],
)
def _sc_gather(table_hbm, idx_hbm, out_hbm, table_v, idx_v, rows, gsems, wsems):
    _gather_body(table_hbm, idx_hbm, out_hbm, table_v, idx_v, rows, gsems, wsems)


def kernel(time, emb_weight, lin_w, lin_b):
    fused = _build_fused_table(emb_weight, lin_w, lin_b)
    idx = time.reshape(B_TOTAL)
    out = _sc_gather(fused, idx)
    return out.reshape(BATCH, HIST, N_HID)
